# Initial kernel scaffold; baseline (speedup 1.0000x reference)
import jax
import jax.numpy as jnp
from jax.experimental import pallas as pl


def kernel(feat, edge_index, W0, b0, W1, b1, W2, b2, Wc, bc):
    raise NotImplementedError("write your pallas kernel here")


# speedup vs baseline: 7.1616x; 7.1616x over previous
import functools

import jax
import jax.numpy as jnp
from jax import lax
from jax.experimental import pallas as pl
from jax.experimental.pallas import tpu as pltpu
from jax.experimental.pallas import tpu_sc as plsc

N = 10000
D = 128
E = 320000
N_CLASSES = 40

NC = 2
NS = 16
NW = NC * NS
EPT = E // NW
C = 80
NCHUNK = EPT // C
ROWS_PT = 624
ROWS_LAST = N - 15 * ROWS_PT

_mesh = plsc.VectorSubcoreMesh(core_axis_name="c", subcore_axis_name="s")


@functools.partial(
    pl.kernel,
    mesh=_mesh,
    out_type=[
        jax.ShapeDtypeStruct((N, D), jnp.float32),
        jax.ShapeDtypeStruct((N, D), jnp.float32),
    ],
    scratch_types=[
        pltpu.VMEM((NCHUNK, C), jnp.int32),
        pltpu.VMEM((NCHUNK, C), jnp.int32),
        pltpu.VMEM((C, D), jnp.float32),
        pltpu.VMEM_SHARED((N, D), jnp.float32),
        pltpu.SemaphoreType.DMA,
    ],
)
def _sc_agg(x_hbm, src_hbm, dst_hbm, zeros_hbm, out0, out1,
            src_idx, dst_idx, rows, acc, sem):
    c = lax.axis_index("c")
    s = lax.axis_index("s")
    wid = c * NS + s
    r0 = s * ROWS_PT
    @pl.when(s < NS - 1)
    def _():
        pltpu.sync_copy(zeros_hbm.at[pl.ds(r0, ROWS_PT)],
                        acc.at[pl.ds(r0, ROWS_PT)])

    @pl.when(s == NS - 1)
    def _():
        pltpu.sync_copy(zeros_hbm.at[pl.ds(r0, ROWS_LAST)],
                        acc.at[pl.ds(r0, ROWS_LAST)])

    pltpu.sync_copy(src_hbm.at[wid], src_idx)
    pltpu.sync_copy(dst_hbm.at[wid], dst_idx)
    plsc.subcore_barrier()

    def body(g, carry):
        pltpu.async_copy(x_hbm.at[src_idx.at[g]], rows, sem).wait()
        pltpu.sync_copy(rows, acc.at[dst_idx.at[g]], add=True)
        return carry

    lax.fori_loop(0, NCHUNK, body, 0)
    plsc.subcore_barrier()

    out = [out0, out1]
    for ci in range(NC):
        @pl.when((c == ci) & (s < NS - 1))
        def _(ci=ci):
            pltpu.sync_copy(acc.at[pl.ds(r0, ROWS_PT)],
                            out[ci].at[pl.ds(r0, ROWS_PT)])

        @pl.when((c == ci) & (s == NS - 1))
        def _(ci=ci):
            pltpu.sync_copy(acc.at[pl.ds(r0, ROWS_LAST)],
                            out[ci].at[pl.ds(r0, ROWS_LAST)])


ROWS_BLK = 1000


def _mlp_body(x_ref, p0_ref, p1_ref, w_ref, b_ref, o_ref):
    z = x_ref[...] + p0_ref[...] + p1_ref[...]
    h = jnp.dot(z, w_ref[...], preferred_element_type=jnp.float32) + b_ref[...]
    o_ref[...] = jnp.maximum(h, 0.0)


def _tc_mlp(x, p0, p1, W, b):
    return pl.pallas_call(
        _mlp_body,
        grid=(N // ROWS_BLK,),
        in_specs=[
            pl.BlockSpec((ROWS_BLK, D), lambda i: (i, 0)),
            pl.BlockSpec((ROWS_BLK, D), lambda i: (i, 0)),
            pl.BlockSpec((ROWS_BLK, D), lambda i: (i, 0)),
            pl.BlockSpec((D, D), lambda i: (0, 0)),
            pl.BlockSpec((1, D), lambda i: (0, 0)),
        ],
        out_specs=pl.BlockSpec((ROWS_BLK, D), lambda i: (i, 0)),
        out_shape=jax.ShapeDtypeStruct((N, D), jnp.float32),
    )(x, p0, p1, W, b.reshape(1, D))


def _final_body(x_ref, p0_ref, p1_ref, w2_ref, b2_ref, wc_ref, bc_ref, o_ref):
    z = x_ref[...] + p0_ref[...] + p1_ref[...]
    h = jnp.dot(z, w2_ref[...], preferred_element_type=jnp.float32) + b2_ref[...]
    h = jnp.maximum(h, 0.0)
    o_ref[...] = jnp.dot(h, wc_ref[...], preferred_element_type=jnp.float32) + bc_ref[...]


def _tc_final(x, p0, p1, W2, b2, Wc, bc):
    return pl.pallas_call(
        _final_body,
        grid=(N // ROWS_BLK,),
        in_specs=[
            pl.BlockSpec((ROWS_BLK, D), lambda i: (i, 0)),
            pl.BlockSpec((ROWS_BLK, D), lambda i: (i, 0)),
            pl.BlockSpec((ROWS_BLK, D), lambda i: (i, 0)),
            pl.BlockSpec((D, D), lambda i: (0, 0)),
            pl.BlockSpec((1, D), lambda i: (0, 0)),
            pl.BlockSpec((D, N_CLASSES), lambda i: (0, 0)),
            pl.BlockSpec((1, N_CLASSES), lambda i: (0, 0)),
        ],
        out_specs=pl.BlockSpec((ROWS_BLK, N_CLASSES), lambda i: (i, 0)),
        out_shape=jax.ShapeDtypeStruct((N, N_CLASSES), jnp.float32),
    )(x, p0, p1, W2, b2.reshape(1, D), Wc, bc.reshape(1, N_CLASSES))


def kernel(feat, edge_index, W0, b0, W1, b1, W2, b2, Wc, bc):
    src = edge_index[0].astype(jnp.int32).reshape(NW, NCHUNK, C)
    dst = edge_index[1].astype(jnp.int32).reshape(NW, NCHUNK, C)
    zeros = jnp.zeros((N, D), jnp.float32)
    p0, p1 = _sc_agg(feat, src, dst, zeros)
    h = _tc_mlp(feat, p0, p1, W0, b0)
    p0, p1 = _sc_agg(h, src, dst, zeros)
    h = _tc_mlp(h, p0, p1, W1, b1)
    p0, p1 = _sc_agg(h, src, dst, zeros)
    return _tc_final(h, p0, p1, W2, b2, Wc, bc)


# speedup vs baseline: 11.4054x; 1.5926x over previous
import functools

import jax
import jax.numpy as jnp
from jax import lax
from jax.experimental import pallas as pl
from jax.experimental.pallas import tpu as pltpu
from jax.experimental.pallas import tpu_sc as plsc

N = 10000
D = 128
E = 320000
N_CLASSES = 40

NC = 2
NS = 16
NW = NC * NS
EPT = E // NW
C = 80
NCHUNK = EPT // C
ROWS_PT = 624
ROWS_LAST = N - 15 * ROWS_PT

_mesh = plsc.VectorSubcoreMesh(core_axis_name="c", subcore_axis_name="s")


@functools.partial(
    pl.kernel,
    mesh=_mesh,
    out_type=[
        jax.ShapeDtypeStruct((N, D), jnp.float32),
        jax.ShapeDtypeStruct((N, D), jnp.float32),
    ],
    scratch_types=[
        pltpu.VMEM((NCHUNK, C), jnp.int32),
        pltpu.VMEM((C,), jnp.int32),
        pltpu.VMEM((C,), jnp.int32),
        pltpu.VMEM((C, D), jnp.float32),
        pltpu.VMEM((C, D), jnp.float32),
        pltpu.VMEM_SHARED((N, D), jnp.float32),
        pltpu.SemaphoreType.DMA,
        pltpu.SemaphoreType.DMA,
        pltpu.SemaphoreType.DMA,
    ],
)
def _sc_agg(x_hbm, src_hbm, dst_hbm, zeros_hbm, out0, out1,
            dst_idx, sidx0, sidx1, rows0, rows1, acc, sem_g, sem_s, sem_i):
    c = lax.axis_index("c")
    s = lax.axis_index("s")
    wid = c * NS + s
    r0 = s * ROWS_PT
    base = wid * EPT

    def _sidx_copy(g, buf):
        return pltpu.make_async_copy(src_hbm.at[pl.ds(base + g * C, C)],
                                     buf, sem_i)

    def _scat_drain():
        pltpu.make_async_copy(rows0, acc.at[dst_idx.at[0]], sem_s).wait()

    @pl.when(s < NS - 1)
    def _():
        pltpu.async_copy(zeros_hbm.at[pl.ds(r0, ROWS_PT)],
                         acc.at[pl.ds(r0, ROWS_PT)], sem_s)

    @pl.when(s == NS - 1)
    def _():
        pltpu.async_copy(zeros_hbm.at[pl.ds(r0, ROWS_LAST)],
                         acc.at[pl.ds(r0, ROWS_LAST)], sem_s)

    _sidx_copy(0, sidx0).start()
    cp_d = pltpu.async_copy(dst_hbm.at[wid], dst_idx, sem_g)
    cp_d.wait()
    _sidx_copy(0, sidx0).wait()

    @pl.when(s < NS - 1)
    def _():
        pltpu.make_async_copy(zeros_hbm.at[pl.ds(r0, ROWS_PT)],
                              acc.at[pl.ds(r0, ROWS_PT)], sem_s).wait()

    @pl.when(s == NS - 1)
    def _():
        pltpu.make_async_copy(zeros_hbm.at[pl.ds(r0, ROWS_LAST)],
                              acc.at[pl.ds(r0, ROWS_LAST)], sem_s).wait()

    plsc.subcore_barrier()

    pltpu.async_copy(x_hbm.at[sidx0], rows0, sem_g)
    _sidx_copy(1, sidx1).start()

    def body(t, carry):
        g0 = 2 * t
        g1 = g0 + 1
        @pl.when(t >= 1)
        def _():
            _scat_drain()
        _sidx_copy(g1, sidx1).wait()
        pltpu.async_copy(x_hbm.at[sidx1], rows1, sem_g)
        pltpu.make_async_copy(x_hbm.at[sidx0], rows0, sem_g).wait()
        _sidx_copy(g0 + 2, sidx0).start()
        pltpu.async_copy(rows0, acc.at[dst_idx.at[g0]], sem_s, add=True)
        @pl.when(t < NCHUNK // 2 - 1)
        def _():
            _scat_drain()
            _sidx_copy(g1 + 1, sidx0).wait()
            pltpu.async_copy(x_hbm.at[sidx0], rows0, sem_g)
        pltpu.make_async_copy(x_hbm.at[sidx1], rows1, sem_g).wait()

        @pl.when(t < NCHUNK // 2 - 1)
        def _():
            _sidx_copy(g1 + 2, sidx1).start()
        pltpu.async_copy(rows1, acc.at[dst_idx.at[g1]], sem_s, add=True)
        return carry

    lax.fori_loop(0, NCHUNK // 2, body, 0)
    _scat_drain()
    _sidx_copy(NCHUNK - 1, sidx0).wait()
    pltpu.async_copy(x_hbm.at[sidx0], rows0, sem_g)
    pltpu.make_async_copy(x_hbm.at[sidx0], rows0, sem_g).wait()
    pltpu.async_copy(rows0, acc.at[dst_idx.at[NCHUNK - 1]], sem_s, add=True)
    _scat_drain()
    _scat_drain()
    plsc.subcore_barrier()

    out = [out0, out1]
    for ci in range(NC):
        @pl.when((c == ci) & (s < NS - 1))
        def _(ci=ci):
            pltpu.sync_copy(acc.at[pl.ds(r0, ROWS_PT)],
                            out[ci].at[pl.ds(r0, ROWS_PT)])

        @pl.when((c == ci) & (s == NS - 1))
        def _(ci=ci):
            pltpu.sync_copy(acc.at[pl.ds(r0, ROWS_LAST)],
                            out[ci].at[pl.ds(r0, ROWS_LAST)])


ROWS_BLK = 1000


def _mlp_body(x_ref, p0_ref, p1_ref, w_ref, b_ref, o_ref):
    z = x_ref[...] + p0_ref[...] + p1_ref[...]
    h = jnp.dot(z, w_ref[...], preferred_element_type=jnp.float32) + b_ref[...]
    o_ref[...] = jnp.maximum(h, 0.0)


def _tc_mlp(x, p0, p1, W, b):
    return pl.pallas_call(
        _mlp_body,
        grid=(N // ROWS_BLK,),
        in_specs=[
            pl.BlockSpec((ROWS_BLK, D), lambda i: (i, 0)),
            pl.BlockSpec((ROWS_BLK, D), lambda i: (i, 0)),
            pl.BlockSpec((ROWS_BLK, D), lambda i: (i, 0)),
            pl.BlockSpec((D, D), lambda i: (0, 0)),
            pl.BlockSpec((1, D), lambda i: (0, 0)),
        ],
        out_specs=pl.BlockSpec((ROWS_BLK, D), lambda i: (i, 0)),
        out_shape=jax.ShapeDtypeStruct((N, D), jnp.float32),
    )(x, p0, p1, W, b.reshape(1, D))


def _final_body(x_ref, p0_ref, p1_ref, w2_ref, b2_ref, wc_ref, bc_ref, o_ref):
    z = x_ref[...] + p0_ref[...] + p1_ref[...]
    h = jnp.dot(z, w2_ref[...], preferred_element_type=jnp.float32) + b2_ref[...]
    h = jnp.maximum(h, 0.0)
    o_ref[...] = jnp.dot(h, wc_ref[...], preferred_element_type=jnp.float32) + bc_ref[...]


def _tc_final(x, p0, p1, W2, b2, Wc, bc):
    return pl.pallas_call(
        _final_body,
        grid=(N // ROWS_BLK,),
        in_specs=[
            pl.BlockSpec((ROWS_BLK, D), lambda i: (i, 0)),
            pl.BlockSpec((ROWS_BLK, D), lambda i: (i, 0)),
            pl.BlockSpec((ROWS_BLK, D), lambda i: (i, 0)),
            pl.BlockSpec((D, D), lambda i: (0, 0)),
            pl.BlockSpec((1, D), lambda i: (0, 0)),
            pl.BlockSpec((D, N_CLASSES), lambda i: (0, 0)),
            pl.BlockSpec((1, N_CLASSES), lambda i: (0, 0)),
        ],
        out_specs=pl.BlockSpec((ROWS_BLK, N_CLASSES), lambda i: (i, 0)),
        out_shape=jax.ShapeDtypeStruct((N, N_CLASSES), jnp.float32),
    )(x, p0, p1, W2, b2.reshape(1, D), Wc, bc.reshape(1, N_CLASSES))


def kernel(feat, edge_index, W0, b0, W1, b1, W2, b2, Wc, bc):
    src = edge_index[0].astype(jnp.int32)
    dst = edge_index[1].astype(jnp.int32).reshape(NW, NCHUNK, C)
    zeros = jnp.zeros((N, D), jnp.float32)
    p0, p1 = _sc_agg(feat, src, dst, zeros)
    h = _tc_mlp(feat, p0, p1, W0, b0)
    p0, p1 = _sc_agg(h, src, dst, zeros)
    h = _tc_mlp(h, p0, p1, W1, b1)
    p0, p1 = _sc_agg(h, src, dst, zeros)
    return _tc_final(h, p0, p1, W2, b2, Wc, bc)


# speedup vs baseline: 11.7264x; 1.0281x over previous
import functools

import jax
import jax.numpy as jnp
from jax import lax
from jax.experimental import pallas as pl
from jax.experimental.pallas import tpu as pltpu
from jax.experimental.pallas import tpu_sc as plsc

N = 10000
D = 128
E = 320000
N_CLASSES = 40

NC = 2
NS = 16
NW = NC * NS
EPT = E // NW
C = 80
NCHUNK = EPT // C
ROWS_PT = 624
ROWS_LAST = N - 15 * ROWS_PT

_mesh = plsc.VectorSubcoreMesh(core_axis_name="c", subcore_axis_name="s")


@functools.partial(
    pl.kernel,
    mesh=_mesh,
    out_type=[
        jax.ShapeDtypeStruct((N, D), jnp.float32),
        jax.ShapeDtypeStruct((N, D), jnp.float32),
    ],
    scratch_types=[
        pltpu.VMEM((NCHUNK, C), jnp.int32),
        pltpu.VMEM((C,), jnp.int32),
        pltpu.VMEM((C,), jnp.int32),
        pltpu.VMEM((C,), jnp.int32),
        pltpu.VMEM((C,), jnp.int32),
        pltpu.VMEM((C, D), jnp.float32),
        pltpu.VMEM((C, D), jnp.float32),
        pltpu.VMEM_SHARED((N, D), jnp.float32),
        pltpu.SemaphoreType.DMA,
        pltpu.SemaphoreType.DMA,
        pltpu.SemaphoreType.DMA,
    ],
)
def _sc_agg(x_hbm, src_hbm, dst_hbm, zeros_hbm, out0, out1,
            dst_idx, si0, si1, si2, si3, rows0, rows1, acc,
            sem_g, sem_s, sem_i):
    c = lax.axis_index("c")
    s = lax.axis_index("s")
    wid = c * NS + s
    r0 = s * ROWS_PT
    base = wid * EPT
    sbufs = [si0, si1, si2, si3]
    rbufs = [rows0, rows1]

    def _sidx_copy(g, buf):
        return pltpu.make_async_copy(src_hbm.at[pl.ds(base + g * C, C)],
                                     buf, sem_i)

    def _scat_drain():
        pltpu.make_async_copy(rows0, acc.at[dst_idx.at[0]], sem_s).wait()

    @pl.when(s < NS - 1)
    def _():
        pltpu.async_copy(zeros_hbm.at[pl.ds(r0, ROWS_PT)],
                         acc.at[pl.ds(r0, ROWS_PT)], sem_s)

    @pl.when(s == NS - 1)
    def _():
        pltpu.async_copy(zeros_hbm.at[pl.ds(r0, ROWS_LAST)],
                         acc.at[pl.ds(r0, ROWS_LAST)], sem_s)

    for j in range(4):
        _sidx_copy(j, sbufs[j]).start()
    cp_d = pltpu.async_copy(dst_hbm.at[wid], dst_idx, sem_g)
    cp_d.wait()
    _sidx_copy(0, si0).wait()
    pltpu.async_copy(x_hbm.at[si0], rows0, sem_g)

    @pl.when(s < NS - 1)
    def _():
        pltpu.make_async_copy(zeros_hbm.at[pl.ds(r0, ROWS_PT)],
                              acc.at[pl.ds(r0, ROWS_PT)], sem_s).wait()

    @pl.when(s == NS - 1)
    def _():
        pltpu.make_async_copy(zeros_hbm.at[pl.ds(r0, ROWS_LAST)],
                              acc.at[pl.ds(r0, ROWS_LAST)], sem_s).wait()

    plsc.subcore_barrier()

    def _chunk_step(g, t, j):
        @pl.when(g >= 1)
        def _():
            _scat_drain()
        _sidx_copy(g + 1, sbufs[(j + 1) % 4]).wait()
        pltpu.async_copy(x_hbm.at[sbufs[(j + 1) % 4]], rbufs[(j + 1) % 2],
                         sem_g)
        pltpu.make_async_copy(x_hbm.at[sbufs[j]], rbufs[j % 2], sem_g).wait()

        @pl.when(g + 4 < NCHUNK)
        def _():
            _sidx_copy(g + 4, sbufs[j]).start()
        pltpu.async_copy(rbufs[j % 2], acc.at[dst_idx.at[g]], sem_s, add=True)

    def body(t, carry):
        for j in range(4):
            _chunk_step(4 * t + j, t, j)
        return carry

    lax.fori_loop(0, NCHUNK // 4, body, 0)
    pltpu.make_async_copy(x_hbm.at[sbufs[0]], rbufs[(NCHUNK - 1) % 2],
                          sem_g).wait()
    pltpu.async_copy(rbufs[(NCHUNK - 1) % 2], acc.at[dst_idx.at[NCHUNK - 1]],
                     sem_s, add=True)
    _scat_drain()
    _scat_drain()
    plsc.subcore_barrier()

    out = [out0, out1]
    for ci in range(NC):
        @pl.when((c == ci) & (s < NS - 1))
        def _(ci=ci):
            pltpu.sync_copy(acc.at[pl.ds(r0, ROWS_PT)],
                            out[ci].at[pl.ds(r0, ROWS_PT)])

        @pl.when((c == ci) & (s == NS - 1))
        def _(ci=ci):
            pltpu.sync_copy(acc.at[pl.ds(r0, ROWS_LAST)],
                            out[ci].at[pl.ds(r0, ROWS_LAST)])


ROWS_BLK = 1000


def _mlp_body(x_ref, p0_ref, p1_ref, w_ref, b_ref, o_ref):
    z = x_ref[...] + p0_ref[...] + p1_ref[...]
    h = jnp.dot(z, w_ref[...], preferred_element_type=jnp.float32) + b_ref[...]
    o_ref[...] = jnp.maximum(h, 0.0)


def _tc_mlp(x, p0, p1, W, b):
    return pl.pallas_call(
        _mlp_body,
        grid=(N // ROWS_BLK,),
        in_specs=[
            pl.BlockSpec((ROWS_BLK, D), lambda i: (i, 0)),
            pl.BlockSpec((ROWS_BLK, D), lambda i: (i, 0)),
            pl.BlockSpec((ROWS_BLK, D), lambda i: (i, 0)),
            pl.BlockSpec((D, D), lambda i: (0, 0)),
            pl.BlockSpec((1, D), lambda i: (0, 0)),
        ],
        out_specs=pl.BlockSpec((ROWS_BLK, D), lambda i: (i, 0)),
        out_shape=jax.ShapeDtypeStruct((N, D), jnp.float32),
    )(x, p0, p1, W, b.reshape(1, D))


def _final_body(x_ref, p0_ref, p1_ref, w2_ref, b2_ref, wc_ref, bc_ref, o_ref):
    z = x_ref[...] + p0_ref[...] + p1_ref[...]
    h = jnp.dot(z, w2_ref[...], preferred_element_type=jnp.float32) + b2_ref[...]
    h = jnp.maximum(h, 0.0)
    o_ref[...] = jnp.dot(h, wc_ref[...], preferred_element_type=jnp.float32) + bc_ref[...]


def _tc_final(x, p0, p1, W2, b2, Wc, bc):
    return pl.pallas_call(
        _final_body,
        grid=(N // ROWS_BLK,),
        in_specs=[
            pl.BlockSpec((ROWS_BLK, D), lambda i: (i, 0)),
            pl.BlockSpec((ROWS_BLK, D), lambda i: (i, 0)),
            pl.BlockSpec((ROWS_BLK, D), lambda i: (i, 0)),
            pl.BlockSpec((D, D), lambda i: (0, 0)),
            pl.BlockSpec((1, D), lambda i: (0, 0)),
            pl.BlockSpec((D, N_CLASSES), lambda i: (0, 0)),
            pl.BlockSpec((1, N_CLASSES), lambda i: (0, 0)),
        ],
        out_specs=pl.BlockSpec((ROWS_BLK, N_CLASSES), lambda i: (i, 0)),
        out_shape=jax.ShapeDtypeStruct((N, N_CLASSES), jnp.float32),
    )(x, p0, p1, W2, b2.reshape(1, D), Wc, bc.reshape(1, N_CLASSES))


def kernel(feat, edge_index, W0, b0, W1, b1, W2, b2, Wc, bc):
    src = edge_index[0].astype(jnp.int32)
    dst = edge_index[1].astype(jnp.int32).reshape(NW, NCHUNK, C)
    zeros = jnp.zeros((N, D), jnp.float32)
    p0, p1 = _sc_agg(feat, src, dst, zeros)
    h = _tc_mlp(feat, p0, p1, W0, b0)
    p0, p1 = _sc_agg(h, src, dst, zeros)
    h = _tc_mlp(h, p0, p1, W1, b1)
    p0, p1 = _sc_agg(h, src, dst, zeros)
    return _tc_final(h, p0, p1, W2, b2, Wc, bc)
